# Initial kernel scaffold; baseline (speedup 1.0000x reference)
import jax
import jax.numpy as jnp
from jax.experimental import pallas as pl


def kernel(input, action_embedding):
    raise NotImplementedError("write your pallas kernel here")


# speedup vs baseline: 1.8748x; 1.8748x over previous
import functools

import jax
import jax.numpy as jnp
from jax import lax
from jax.experimental import pallas as pl
from jax.experimental.pallas import tpu as pltpu
from jax.experimental.pallas import tpu_sc as plsc

D = 64
IDX_W = 128
NC, NS = 2, 16
NW = NC * NS
B_TOTAL = 16384 * 50
N_ROWS = B_TOTAL // IDX_W
ROWS_W = N_ROWS // NW
CH_ROWS = 4
CH = CH_ROWS * IDX_W
N_IT = ROWS_W // CH_ROWS


def _body(idx_hbm, table_hbm, out_hbm, idx_all, rows_v, gsem0, gsem1):
    wid = lax.axis_index("s") * NC + lax.axis_index("c")
    base_row = wid * ROWS_W
    base_out = wid * (ROWS_W * IDX_W)

    pltpu.sync_copy(idx_hbm.at[pl.ds(base_row, ROWS_W)], idx_all)

    gsems = (gsem0, gsem1)

    def fire(it, s):
        for j in range(CH_ROWS):
            pltpu.async_copy(
                table_hbm.at[idx_all.at[it * CH_ROWS + j]],
                rows_v.at[s, pl.ds(j * IDX_W, IDX_W)],
                gsems[s])

    fire(0, 0)
    fire(1, 1)

    @pl.loop(0, N_IT, step=2)
    def _(i):
        for s in range(2):
            it = i + s
            for j in range(CH_ROWS):
                pltpu.make_async_copy(
                    table_hbm.at[idx_all.at[j]],
                    rows_v.at[s, pl.ds(j * IDX_W, IDX_W)],
                    gsems[s]).wait()
            pltpu.sync_copy(rows_v.at[s],
                            out_hbm.at[pl.ds(base_out + it * CH, CH)])

            @pl.when(it + 2 < N_IT)
            def _():
                fire(it + 2, s)


@jax.jit
def _gather(idx2d, table):
    mesh = plsc.VectorSubcoreMesh(core_axis_name="c", subcore_axis_name="s")
    f = functools.partial(
        pl.kernel,
        mesh=mesh,
        out_type=jax.ShapeDtypeStruct((B_TOTAL, D), jnp.float32),
        scratch_types=[
            pltpu.VMEM((ROWS_W, IDX_W), jnp.int32),
            pltpu.VMEM((2, CH, D), jnp.float32),
            pltpu.SemaphoreType.DMA,
            pltpu.SemaphoreType.DMA,
        ],
        compiler_params=pltpu.CompilerParams(use_tc_tiling_on_sc=False),
    )(_body)
    return f(idx2d, table)


def kernel(input, action_embedding):
    idx2d = input.reshape(N_ROWS, IDX_W).astype(jnp.int32)
    out = _gather(idx2d, action_embedding)
    return out.reshape(input.shape + (D,))
